# baseline (speedup 1.0000x reference)
import functools

import jax
import jax.numpy as jnp
from jax import lax
from jax.experimental import pallas as pl
from jax.experimental.pallas import tpu as pltpu
from jax.experimental.pallas import tpu_sc as plsc


def _make_gather(n_rows: int, vocab: int, d: int):
    info = plsc.get_sparse_core_info()
    nc, ns = info.num_cores, info.num_subcores
    nw = nc * ns
    assert n_rows % nw == 0
    b_per_w = n_rows // nw
    ch = 512
    assert b_per_w % ch == 0
    n_chunks = b_per_w // ch

    @functools.partial(
        pl.kernel,
        mesh=plsc.VectorSubcoreMesh(core_axis_name="c", subcore_axis_name="s"),
        out_type=jax.ShapeDtypeStruct((n_rows, d), jnp.float32),
        compiler_params=pltpu.CompilerParams(use_tc_tiling_on_sc=False),
        scratch_types=[
            pltpu.VMEM((ch,), jnp.int32),
            pltpu.VMEM((ch, d), jnp.float32),
            pltpu.SemaphoreType.DMA,
        ],
    )
    def gather(table_hbm, idx_hbm, out_hbm, idx_v, rows_v, sem):
        wid = lax.axis_index("s") * nc + lax.axis_index("c")
        base = wid * b_per_w

        def chunk(i, carry):
            off = base + i * ch
            pltpu.sync_copy(idx_hbm.at[pl.ds(off, ch)], idx_v)
            pltpu.async_copy(table_hbm.at[idx_v], rows_v, sem).wait()
            pltpu.sync_copy(rows_v, out_hbm.at[pl.ds(off, ch)])
            return carry

        lax.fori_loop(0, n_chunks, chunk, 0)

    return gather


def kernel(features, word_embeddings_weight):
    b, s = features.shape
    v, d = word_embeddings_weight.shape
    n = b * s
    idx = features.reshape(n).astype(jnp.int32)
    gather = _make_gather(n, v, d)
    out = gather(word_embeddings_weight, idx)
    return out.reshape(b, s, d)


# speedup vs baseline: 1.0483x; 1.0483x over previous
import functools

import jax
import jax.numpy as jnp
from jax import lax
from jax.experimental import pallas as pl
from jax.experimental.pallas import tpu as pltpu
from jax.experimental.pallas import tpu_sc as plsc


def _make_gather(n_rows: int, d: int):
    info = plsc.get_sparse_core_info()
    nc, ns = info.num_cores, info.num_subcores
    nw = nc * ns
    assert n_rows % nw == 0
    b_per_w = n_rows // nw
    ch = 400
    nbuf = 4
    la = 2
    assert b_per_w % (ch * nbuf) == 0
    n_chunks = b_per_w // ch
    n_rounds = n_chunks // nbuf

    @functools.partial(
        pl.kernel,
        mesh=plsc.VectorSubcoreMesh(core_axis_name="c", subcore_axis_name="s"),
        out_type=jax.ShapeDtypeStruct((n_rows, d), jnp.float32),
        compiler_params=pltpu.CompilerParams(use_tc_tiling_on_sc=False),
        scratch_types=[
            pltpu.VMEM((b_per_w,), jnp.int32),
            *[pltpu.VMEM((ch, d), jnp.float32) for _ in range(nbuf)],
            pltpu.SemaphoreType.DMA((nbuf,)),
            pltpu.SemaphoreType.DMA((nbuf,)),
        ],
    )
    def gather(table_hbm, idx_hbm, out_hbm, idx_v, r0, r1, r2, r3, sg, sw):
        rows = (r0, r1, r2, r3)
        wid = lax.axis_index("s") * nc + lax.axis_index("c")
        base = wid * b_per_w
        pltpu.sync_copy(idx_hbm.at[pl.ds(base, b_per_w)], idx_v)

        def start_gather(c, b):
            pltpu.make_async_copy(
                table_hbm.at[idx_v.at[pl.ds(c * ch, ch)]], rows[b], sg.at[b]
            ).start()

        def wait_gather(b):
            pltpu.make_async_copy(
                table_hbm.at[idx_v.at[pl.ds(0, ch)]], rows[b], sg.at[b]
            ).wait()

        def start_wb(c, b):
            pltpu.make_async_copy(
                rows[b], out_hbm.at[pl.ds(base + c * ch, ch)], sw.at[b]
            ).start()

        def wait_wb(b):
            pltpu.make_async_copy(
                rows[b], out_hbm.at[pl.ds(base, ch)], sw.at[b]
            ).wait()

        for b in range(la):
            start_gather(b, b)

        def round_body(r, carry):
            for b in range(nbuf):
                c = r * nbuf + b
                b2 = (b + la) % nbuf
                wait_gather(b)
                start_wb(c, b)

                @pl.when(c + la < n_chunks)
                def _():
                    @pl.when(c >= nbuf - la)
                    def _():
                        wait_wb(b2)

                    start_gather(c + la, b2)

            return carry

        lax.fori_loop(0, n_rounds, round_body, 0)

        for b in range(nbuf - la):
            wait_wb((n_chunks - (nbuf - la) + b) % nbuf)

    return gather


def kernel(features, word_embeddings_weight):
    b, s = features.shape
    v, d = word_embeddings_weight.shape
    n = b * s
    idx = features.reshape(n).astype(jnp.int32)
    gather = _make_gather(n, d)
    out = gather(word_embeddings_weight, idx)
    return out.reshape(b, s, d)
